# baseline (speedup 1.0000x reference)
import functools

import jax
import jax.numpy as jnp
from jax.experimental import pallas as pl
from jax.experimental.pallas import tpu as pltpu

Q = 1024
D = 64
KB = 2000


def _nn_kernel(q_ref, k_ref, out_ref, minval_ref, minidx_ref, *, nsteps, kb):
    step = pl.program_id(0)

    @pl.when(step == 0)
    def _init():
        minval_ref[...] = jnp.full(minval_ref.shape, jnp.inf, jnp.float32)
        minidx_ref[...] = jnp.zeros(minidx_ref.shape, jnp.int32)

    q = q_ref[...]
    k = k_ref[...]
    qk = jax.lax.dot_general(q, k, (((1,), (1,)), ((), ())),
                             preferred_element_type=jnp.float32)
    q_sq = jnp.sum(q * q, axis=1, keepdims=True)
    k_sq = jnp.sum(k * k, axis=1)
    d2 = (q_sq - 2.0 * qk) + k_sq[None, :]
    bmin = jnp.min(d2, axis=1, keepdims=True)
    iota = jax.lax.broadcasted_iota(jnp.int32, d2.shape, 1)
    bidx = jnp.min(jnp.where(d2 == bmin, iota, jnp.int32(2**30)),
                   axis=1, keepdims=True)
    bidx = bidx + step * kb
    better = bmin < minval_ref[...]
    minval_ref[...] = jnp.where(better, bmin, minval_ref[...])
    minidx_ref[...] = jnp.where(better, bidx, minidx_ref[...])

    @pl.when(step == nsteps - 1)
    def _done():
        out_ref[...] = minidx_ref[...]


def kernel(queries, keys):
    n_keys = keys.shape[0]
    nsteps = n_keys // KB
    out = pl.pallas_call(
        functools.partial(_nn_kernel, nsteps=nsteps, kb=KB),
        grid=(nsteps,),
        in_specs=[
            pl.BlockSpec((Q, D), lambda i: (0, 0)),
            pl.BlockSpec((KB, D), lambda i: (i, 0)),
        ],
        out_specs=pl.BlockSpec((Q, 1), lambda i: (0, 0)),
        out_shape=jax.ShapeDtypeStruct((Q, 1), jnp.int32),
        scratch_shapes=[
            pltpu.VMEM((Q, 1), jnp.float32),
            pltpu.VMEM((Q, 1), jnp.int32),
        ],
    )(queries, keys)
    return out.reshape(Q)


# speedup vs baseline: 1.1405x; 1.1405x over previous
import functools

import jax
import jax.numpy as jnp
from jax.experimental import pallas as pl
from jax.experimental.pallas import tpu as pltpu

Q = 1024
D = 64
KB = 2000


def _nn_kernel(q_ref, k_ref, out_ref, minval_ref, minidx_ref, *, nsteps, kb):
    step = pl.program_id(0)

    @pl.when(step == 0)
    def _init():
        minval_ref[...] = jnp.full(minval_ref.shape, jnp.inf, jnp.float32)
        minidx_ref[...] = jnp.zeros(minidx_ref.shape, jnp.int32)

    q = q_ref[...]
    k = k_ref[...]
    qk2 = jax.lax.dot_general(q * (-2.0), k, (((1,), (1,)), ((), ())),
                              preferred_element_type=jnp.float32)
    q_sq = jnp.sum(q * q, axis=1, keepdims=True)
    k_sq = jnp.sum(k * k, axis=1)
    d2 = (q_sq + qk2) + k_sq[None, :]
    bmin = jnp.min(d2, axis=1, keepdims=True)
    iota = jax.lax.broadcasted_iota(jnp.int32, d2.shape, 1).astype(jnp.float32)
    bidx_f = jnp.min(jnp.where(d2 == bmin, iota, jnp.float32(1e9)),
                     axis=1, keepdims=True)
    bidx = bidx_f.astype(jnp.int32) + step * kb
    better = bmin < minval_ref[...]
    minval_ref[...] = jnp.where(better, bmin, minval_ref[...])
    minidx_ref[...] = jnp.where(better, bidx, minidx_ref[...])

    @pl.when(step == nsteps - 1)
    def _done():
        out_ref[...] = minidx_ref[...]


def kernel(queries, keys):
    n_keys = keys.shape[0]
    nsteps = n_keys // KB
    out = pl.pallas_call(
        functools.partial(_nn_kernel, nsteps=nsteps, kb=KB),
        grid=(nsteps,),
        in_specs=[
            pl.BlockSpec((Q, D), lambda i: (0, 0)),
            pl.BlockSpec((KB, D), lambda i: (i, 0)),
        ],
        out_specs=pl.BlockSpec((Q, 1), lambda i: (0, 0)),
        out_shape=jax.ShapeDtypeStruct((Q, 1), jnp.int32),
        scratch_shapes=[
            pltpu.VMEM((Q, 1), jnp.float32),
            pltpu.VMEM((Q, 1), jnp.int32),
        ],
    )(queries, keys)
    return out.reshape(Q)


# speedup vs baseline: 1.4288x; 1.2528x over previous
import functools

import jax
import jax.numpy as jnp
from jax.experimental import pallas as pl
from jax.experimental.pallas import tpu as pltpu

Q = 1024
D = 64
KB = 2048
RC = 1024
LW = 128


def _nn_kernel(q_ref, k_ref, out_ref, minval_ref, minidx_ref, *, nsteps, n_keys):
    step = pl.program_id(0)

    @pl.when(step == 0)
    def _init():
        minval_ref[...] = jnp.full(minval_ref.shape, jnp.inf, jnp.float32)
        minidx_ref[...] = jnp.zeros(minidx_ref.shape, jnp.float32)

    k = k_ref[...]
    k_sq = jnp.sum(k * k, axis=1)[None, :]
    lane1 = jax.lax.broadcasted_iota(jnp.int32, (1, KB), 1)
    valid = (step * KB + lane1) < n_keys
    k_sq = jnp.where(valid, k_sq, jnp.inf)

    ncols = KB // LW
    base0 = (step * KB).astype(jnp.float32)

    for rc in range(Q // RC):
        rows = pl.ds(rc * RC, RC)
        q = q_ref[rows, :]
        q_sq = jnp.sum(q * q, axis=1, keepdims=True)
        qm = q * (-2.0)
        qk2 = jax.lax.dot_general(qm, k, (((1,), (1,)), ((), ())),
                                  preferred_element_type=jnp.float32)
        m = minval_ref[rows, :]
        idx = minidx_ref[rows, :]
        for c in range(ncols):
            tile = jax.lax.slice(qk2, (0, c * LW), (RC, (c + 1) * LW))
            ksl = jax.lax.slice(k_sq, (0, c * LW), (1, (c + 1) * LW))
            d2 = (q_sq + tile) + ksl
            upd = d2 < m
            m = jnp.where(upd, d2, m)
            idx = jnp.where(upd, base0 + jnp.float32(c * LW), idx)
        minval_ref[rows, :] = m
        minidx_ref[rows, :] = idx

    @pl.when(step == nsteps - 1)
    def _done():
        lane_f = jax.lax.broadcasted_iota(jnp.int32, (Q, LW), 1).astype(jnp.float32)
        m = minval_ref[...]
        idx = minidx_ref[...]
        bmin = jnp.min(m, axis=1, keepdims=True)
        cand = jnp.where(m == bmin, idx + lane_f, jnp.float32(3e7))
        out_ref[...] = jnp.min(cand, axis=1, keepdims=True).astype(jnp.int32)


def kernel(queries, keys):
    n_keys = keys.shape[0]
    nsteps = (n_keys + KB - 1) // KB
    out = pl.pallas_call(
        functools.partial(_nn_kernel, nsteps=nsteps, n_keys=n_keys),
        grid=(nsteps,),
        in_specs=[
            pl.BlockSpec((Q, D), lambda i: (0, 0)),
            pl.BlockSpec((KB, D), lambda i: (i, 0)),
        ],
        out_specs=pl.BlockSpec((Q, 1), lambda i: (0, 0)),
        out_shape=jax.ShapeDtypeStruct((Q, 1), jnp.int32),
        scratch_shapes=[
            pltpu.VMEM((Q, LW), jnp.float32),
            pltpu.VMEM((Q, LW), jnp.float32),
        ],
    )(queries, keys)
    return out.reshape(Q)


# speedup vs baseline: 1.4624x; 1.0235x over previous
import functools

import jax
import jax.numpy as jnp
from jax.experimental import pallas as pl
from jax.experimental.pallas import tpu as pltpu

Q = 1024
D = 64
KB = 4096
RC = 1024
LW = 128


def _nn_kernel(q_ref, k_ref, out_ref, minval_ref, minidx_ref, mineven_ref,
               qm_ref, qsq_ref, *, nsteps, n_keys):
    step = pl.program_id(0)

    @pl.when(step == 0)
    def _init():
        minval_ref[...] = jnp.full(minval_ref.shape, jnp.inf, jnp.float32)
        minidx_ref[...] = jnp.zeros(minidx_ref.shape, jnp.float32)
        mineven_ref[...] = jnp.zeros(mineven_ref.shape, jnp.float32)
        qa = q_ref[...]
        qm_ref[...] = qa * (-2.0)
        qsq_ref[...] = jnp.sum(qa * qa, axis=1, keepdims=True)

    rowk = jax.lax.broadcasted_iota(jnp.int32, (KB, 1), 0)
    k = jnp.where((step * KB + rowk) < n_keys, k_ref[...], 0.0)
    k_sq = jnp.sum(k * k, axis=1)[None, :]
    lane1 = jax.lax.broadcasted_iota(jnp.int32, (1, KB), 1)
    valid = (step * KB + lane1) < n_keys
    k_sq = jnp.where(valid, k_sq, jnp.inf)

    ncols = KB // LW
    base0 = (step * KB).astype(jnp.float32)

    for rc in range(Q // RC):
        rows = pl.ds(rc * RC, RC)
        q_sq = qsq_ref[rows, :]
        qm = qm_ref[rows, :]
        qk2 = jax.lax.dot_general(qm, k, (((1,), (1,)), ((), ())),
                                  preferred_element_type=jnp.float32)
        m = minval_ref[rows, :]
        idx = minidx_ref[rows, :]
        weven = mineven_ref[rows, :]
        for c in range(0, ncols, 2):
            ta = jax.lax.slice(qk2, (0, c * LW), (RC, (c + 1) * LW))
            tb = jax.lax.slice(qk2, (0, (c + 1) * LW), (RC, (c + 2) * LW))
            ka = jax.lax.slice(k_sq, (0, c * LW), (1, (c + 1) * LW))
            kb = jax.lax.slice(k_sq, (0, (c + 1) * LW), (1, (c + 2) * LW))
            d2a = (q_sq + ta) + ka
            d2b = (q_sq + tb) + kb
            mp = jnp.minimum(d2a, d2b)
            upd = mp < m
            m = jnp.where(upd, mp, m)
            idx = jnp.where(upd, base0 + jnp.float32(c * LW), idx)
            weven = jnp.where(upd, d2a, weven)
        minval_ref[rows, :] = m
        minidx_ref[rows, :] = idx
        mineven_ref[rows, :] = weven

    @pl.when(step == nsteps - 1)
    def _done():
        lane_f = jax.lax.broadcasted_iota(jnp.int32, (Q, LW), 1).astype(jnp.float32)
        m = minval_ref[...]
        idx = minidx_ref[...]
        weven = mineven_ref[...]
        off = jnp.where(weven == m, jnp.float32(0.0), jnp.float32(LW))
        gidx = (idx + off) + lane_f
        bmin = jnp.min(m, axis=1, keepdims=True)
        cand = jnp.where(m == bmin, gidx, jnp.float32(3e7))
        out_ref[...] = jnp.min(cand, axis=1, keepdims=True).astype(jnp.int32)


def kernel(queries, keys):
    n_keys = keys.shape[0]
    nsteps = (n_keys + KB - 1) // KB
    out = pl.pallas_call(
        functools.partial(_nn_kernel, nsteps=nsteps, n_keys=n_keys),
        grid=(nsteps,),
        in_specs=[
            pl.BlockSpec((Q, D), lambda i: (0, 0)),
            pl.BlockSpec((KB, D), lambda i: (i, 0)),
        ],
        out_specs=pl.BlockSpec((Q, 1), lambda i: (0, 0)),
        out_shape=jax.ShapeDtypeStruct((Q, 1), jnp.int32),
        scratch_shapes=[
            pltpu.VMEM((Q, LW), jnp.float32),
            pltpu.VMEM((Q, LW), jnp.float32),
            pltpu.VMEM((Q, LW), jnp.float32),
            pltpu.VMEM((Q, D), jnp.float32),
            pltpu.VMEM((Q, 1), jnp.float32),
        ],
    )(queries, keys)
    return out.reshape(Q)
